# baseline (speedup 1.0000x reference)
import jax
import jax.numpy as jnp
from jax.experimental import pallas as pl
from jax.experimental.pallas import tpu as pltpu

NBUF = 3
BLK = 1024
NSPLIT = 2
HALF = BLK // NSPLIT


def _pipelined_kernel(h_hbm, w_vmem, o_hbm, inbuf, outbuf, scale_buf, in_sems, out_sems):
    num_tokens = h_hbm.shape[0]
    nblocks = num_tokens // BLK

    w = w_vmem[...]
    scale_buf[...] = jnp.sum(w, axis=1, keepdims=True)

    def in_copy(t, slot, part):
        return pltpu.make_async_copy(
            h_hbm.at[pl.ds(t * BLK + part * HALF, HALF), :],
            inbuf.at[slot, pl.ds(part * HALF, HALF), :],
            in_sems.at[slot, part],
        )

    def out_copy(t, slot, part):
        return pltpu.make_async_copy(
            outbuf.at[slot, pl.ds(part * HALF, HALF), :],
            o_hbm.at[pl.ds(t * BLK + part * HALF, HALF), :],
            out_sems.at[slot, part],
        )

    for s in range(NBUF):
        for p in range(NSPLIT):
            in_copy(s, s, p).start()

    def body(t, _):
        slot = jax.lax.rem(t, NBUF)
        for p in range(NSPLIT):
            in_copy(t, slot, p).wait()

        @pl.when(t >= NBUF)
        def _():
            for p in range(NSPLIT):
                out_copy(t - NBUF, slot, p).wait()

        s = scale_buf[pl.ds(t * BLK, BLK), :]
        outbuf[slot] = inbuf[slot] * s
        for p in range(NSPLIT):
            out_copy(t, slot, p).start()

        @pl.when(t + NBUF < nblocks)
        def _():
            for p in range(NSPLIT):
                in_copy(t + NBUF, slot, p).start()

        return 0

    jax.lax.fori_loop(0, nblocks, body, 0)

    for s in range(NBUF):
        t = nblocks - NBUF + s
        for p in range(NSPLIT):
            out_copy(t, jax.lax.rem(jnp.int32(t), NBUF), p).wait()


def kernel(hidden_states, routing_indices, routing_weights):
    del routing_indices
    num_tokens, hidden_dim = hidden_states.shape

    return pl.pallas_call(
        _pipelined_kernel,
        in_specs=[
            pl.BlockSpec(memory_space=pltpu.MemorySpace.HBM),
            pl.BlockSpec(memory_space=pltpu.VMEM),
        ],
        out_specs=pl.BlockSpec(memory_space=pltpu.MemorySpace.HBM),
        out_shape=jax.ShapeDtypeStruct((num_tokens, hidden_dim), hidden_states.dtype),
        scratch_shapes=[
            pltpu.VMEM((NBUF, BLK, hidden_dim), hidden_states.dtype),
            pltpu.VMEM((NBUF, BLK, hidden_dim), hidden_states.dtype),
            pltpu.VMEM((num_tokens, 1), jnp.float32),
            pltpu.SemaphoreType.DMA((NBUF, NSPLIT)),
            pltpu.SemaphoreType.DMA((NBUF, NSPLIT)),
        ],
    )(hidden_states, routing_weights)
